# Initial kernel scaffold; baseline (speedup 1.0000x reference)
import jax
import jax.numpy as jnp
from jax.experimental import pallas as pl


def kernel(task_state_scheduled, task_state_ready, task_lengths, vm_completion_times, vm_speeds, vm_energy_rates, params, task_assignments, compatibilities, task_dependencies):
    raise NotImplementedError("write your pallas kernel here")


# speedup vs baseline: 4.3691x; 4.3691x over previous
import functools

import jax
import jax.numpy as jnp
from jax import lax
from jax.experimental import pallas as pl
from jax.experimental.pallas import tpu as pltpu
from jax.experimental.pallas import tpu_sc as plsc

H = 512
D = 128
NT = 8000
NV = 2000
N = NT + NV
E = 320000
NW = 32
C = 128
KCH = 79
PW = KCH * C
EP = NW * PW
NPAD = 10112
ZR = NPAD // 16
OR = 624
RB = 1000
GRID = N // RB

_MESH = plsc.VectorSubcoreMesh(core_axis_name="c", subcore_axis_name="s")


def _sc_segsum(xflat, srcq, dst3, zrows, q_chunks):
    @functools.partial(
        pl.kernel,
        out_type=jax.ShapeDtypeStruct((2, q_chunks, N, 128), jnp.float32),
        mesh=_MESH,
        scratch_types=[
            pltpu.VMEM((KCH, C), jnp.int32),
            pltpu.VMEM((KCH, C), jnp.int32),
            pltpu.VMEM((C, 128), jnp.float32),
            pltpu.VMEM_SHARED((NPAD, 128), jnp.float32),
            pltpu.SemaphoreType.DMA((2,)),
        ],
    )
    def k(x_hbm, srcq_hbm, dst_hbm, z_hbm, out_hbm, src_v, dst_v, rows, acc, sem):
        c = lax.axis_index("c")
        s = lax.axis_index("s")
        w = c * 16 + s
        for q in range(q_chunks):
            pltpu.sync_copy(srcq_hbm.at[q, w], src_v)
            pltpu.sync_copy(dst_hbm.at[w], dst_v)
            pltpu.sync_copy(z_hbm, acc.at[pl.ds(s * ZR, ZR)])
            plsc.subcore_barrier()

            def body(kk, carry):
                pltpu.async_copy(x_hbm.at[src_v.at[kk]], rows, sem.at[0]).wait()
                pltpu.sync_copy(rows, acc.at[dst_v.at[kk]], add=True)
                return carry

            lax.fori_loop(0, KCH, body, 0)
            plsc.subcore_barrier()
            pltpu.sync_copy(
                acc.at[pl.ds(s * OR, OR)], out_hbm.at[c, q, pl.ds(s * OR, OR)]
            )

            @pl.when(s == 15)
            def _():
                pltpu.sync_copy(
                    acc.at[pl.ds(16 * OR, N - 16 * OR)],
                    out_hbm.at[c, q, pl.ds(16 * OR, N - 16 * OR)],
                )

            plsc.subcore_barrier()

    return k(xflat, srcq, dst3, zrows)


def _sc_edge_embed(ne, src3, dst3):
    @functools.partial(
        pl.kernel,
        out_type=jax.ShapeDtypeStruct((E, 256), jnp.float32),
        mesh=_MESH,
        scratch_types=[
            pltpu.VMEM((KCH, C), jnp.int32),
            pltpu.VMEM((KCH, C), jnp.int32),
            pltpu.VMEM((2, C, 128), jnp.float32),
            pltpu.SemaphoreType.DMA((2,)),
        ],
    )
    def k(ne_hbm, src_hbm, dst_hbm, out_hbm, srcb, dstb, rows, sem):
        c = lax.axis_index("c")
        s = lax.axis_index("s")
        w = c * 16 + s
        base = w * PW
        pltpu.sync_copy(src_hbm.at[w], srcb)
        pltpu.sync_copy(dst_hbm.at[w], dstb)

        def body(kk, carry):
            off = base + kk * C

            @pl.when(off + C <= E)
            def _():
                pltpu.async_copy(
                    ne_hbm.at[srcb.at[kk]], rows.at[0], sem.at[0]
                ).wait()
                pltpu.sync_copy(rows.at[0], out_hbm.at[pl.ds(off, C), pl.ds(0, 128)])
                pltpu.async_copy(
                    ne_hbm.at[dstb.at[kk]], rows.at[1], sem.at[1]
                ).wait()
                pltpu.sync_copy(rows.at[1], out_hbm.at[pl.ds(off, C), pl.ds(128, 128)])

            return carry

        lax.fori_loop(0, KCH, body, 0)

    return k(ne, src3, dst3)


def _encoder(x8, W1, b1, g1, be1, W2, b2, g2, be2, W3, b3):
    n = x8.shape[0]

    def body(x_r, w1_r, b1_r, g1_r, be1_r, w2_r, b2_r, g2_r, be2_r, w3_r, b3_r, o_r):
        def bn_relu(h, g, be):
            mu = jnp.mean(h, axis=0, keepdims=True)
            var = jnp.mean((h - mu) ** 2, axis=0, keepdims=True)
            return jax.nn.relu((h - mu) / jnp.sqrt(var + 1e-5) * g + be)

        h = jnp.dot(x_r[...], w1_r[...], preferred_element_type=jnp.float32) + b1_r[...]
        h = bn_relu(h, g1_r[...], be1_r[...])
        h = jnp.dot(h, w2_r[...], preferred_element_type=jnp.float32) + b2_r[...]
        h = bn_relu(h, g2_r[...], be2_r[...])
        o_r[...] = jnp.dot(h, w3_r[...], preferred_element_type=jnp.float32) + b3_r[...]

    return pl.pallas_call(
        body, out_shape=jax.ShapeDtypeStruct((n, D), jnp.float32)
    )(x8, W1, b1, g1, be1, W2, b2, g2, be2, W3, b3)


def _conv1(x, parts, Wa, ba, Wb, bb):
    def body(x_r, p_r, wa_r, ba_r, wb_r, bb_r, o_r):
        h = x_r[...] + p_r[0] + p_r[1]
        t = jax.nn.relu(jnp.dot(h, wa_r[...], preferred_element_type=jnp.float32) + ba_r[...])
        o = jax.nn.relu(jnp.dot(t, wb_r[...], preferred_element_type=jnp.float32) + bb_r[...])
        for q in range(4):
            o_r[q] = o[:, 128 * q:128 * (q + 1)]

    return pl.pallas_call(
        body,
        grid=(GRID,),
        in_specs=[
            pl.BlockSpec((RB, 128), lambda i: (i, 0)),
            pl.BlockSpec((2, RB, 128), lambda i: (0, i, 0)),
            pl.BlockSpec((128, H), lambda i: (0, 0)),
            pl.BlockSpec((1, H), lambda i: (0, 0)),
            pl.BlockSpec((H, H), lambda i: (0, 0)),
            pl.BlockSpec((1, H), lambda i: (0, 0)),
        ],
        out_specs=pl.BlockSpec((4, RB, 128), lambda i: (0, i, 0)),
        out_shape=jax.ShapeDtypeStruct((4, N, 128), jnp.float32),
    )(x, parts, Wa, ba, Wb, bb)


def _conv2_proj(h1q, parts, Wa, ba, Wb, bb, W3a):
    def body(hq_r, p_r, wa_r, ba_r, wb_r, bb_r, w3a_r, y_r):
        t = jnp.zeros((RB, H), jnp.float32)
        for q in range(4):
            hq = hq_r[q] + p_r[0, q] + p_r[1, q]
            t = t + jnp.dot(hq, wa_r[128 * q:128 * (q + 1), :],
                            preferred_element_type=jnp.float32)
        t = jax.nn.relu(t + ba_r[...])
        h2 = jax.nn.relu(jnp.dot(t, wb_r[...], preferred_element_type=jnp.float32) + bb_r[...])
        y_r[...] = jnp.dot(h2, w3a_r[...], preferred_element_type=jnp.float32)

    return pl.pallas_call(
        body,
        grid=(GRID,),
        in_specs=[
            pl.BlockSpec((4, RB, 128), lambda i: (0, i, 0)),
            pl.BlockSpec((2, 4, RB, 128), lambda i: (0, 0, i, 0)),
            pl.BlockSpec((H, H), lambda i: (0, 0)),
            pl.BlockSpec((1, H), lambda i: (0, 0)),
            pl.BlockSpec((H, H), lambda i: (0, 0)),
            pl.BlockSpec((1, H), lambda i: (0, 0)),
            pl.BlockSpec((H, 128), lambda i: (0, 0)),
        ],
        out_specs=pl.BlockSpec((RB, 128), lambda i: (i, 0)),
        out_shape=jax.ShapeDtypeStruct((N, 128), jnp.float32),
    )(h1q, parts, Wa, ba, Wb, bb, W3a)


def _conv3_out(y, parts, b3a, W3b, b3b):
    def body(y_r, p_r, ba_r, wb_r, bb_r, ne_r, g_r):
        i = pl.program_id(0)
        t = jax.nn.relu(y_r[...] + p_r[0] + p_r[1] + ba_r[...])
        ne = jnp.dot(t, wb_r[...], preferred_element_type=jnp.float32) + bb_r[...]
        ne_r[...] = ne
        ssum = jnp.sum(ne, axis=0, keepdims=True)

        @pl.when(i == 0)
        def _():
            g_r[...] = ssum

        @pl.when(i > 0)
        def _():
            g_r[...] = g_r[...] + ssum

        @pl.when(i == GRID - 1)
        def _():
            g_r[...] = g_r[...] * (1.0 / N)

    return pl.pallas_call(
        body,
        grid=(GRID,),
        in_specs=[
            pl.BlockSpec((RB, 128), lambda i: (i, 0)),
            pl.BlockSpec((2, RB, 128), lambda i: (0, i, 0)),
            pl.BlockSpec((1, 128), lambda i: (0, 0)),
            pl.BlockSpec((128, 128), lambda i: (0, 0)),
            pl.BlockSpec((1, 128), lambda i: (0, 0)),
        ],
        out_specs=[
            pl.BlockSpec((RB, 128), lambda i: (i, 0)),
            pl.BlockSpec((1, 128), lambda i: (0, 0)),
        ],
        out_shape=[
            jax.ShapeDtypeStruct((N, 128), jnp.float32),
            jax.ShapeDtypeStruct((1, 128), jnp.float32),
        ],
    )(y, parts, b3a, W3b, b3b)


def kernel(task_state_scheduled, task_state_ready, task_lengths,
           vm_completion_times, vm_speeds, vm_energy_rates, params,
           task_assignments, compatibilities, task_dependencies):
    p = params
    r2 = lambda v: v.reshape(1, -1)

    task_x = jnp.stack([task_state_scheduled, task_state_ready, task_lengths], axis=-1)
    task_x = jnp.pad(task_x, ((0, 0), (0, 5)))
    vm_x = jnp.stack([vm_completion_times, vm_speeds, vm_energy_rates], axis=-1)
    vm_x = jnp.pad(vm_x, ((0, 0), (0, 5)))

    task_h = _encoder(task_x, jnp.pad(p['task_W1'], ((0, 5), (0, 0))),
                      r2(p['task_b1']), r2(p['task_g1']), r2(p['task_be1']),
                      p['task_W2'], r2(p['task_b2']), r2(p['task_g2']), r2(p['task_be2']),
                      p['task_W3'], r2(p['task_b3']))
    vm_h = _encoder(vm_x, jnp.pad(p['vm_W1'], ((0, 5), (0, 0))),
                    r2(p['vm_b1']), r2(p['vm_g1']), r2(p['vm_be1']),
                    p['vm_W2'], r2(p['vm_b2']), r2(p['vm_g2']), r2(p['vm_be2']),
                    p['vm_W3'], r2(p['vm_b3']))
    node_x = jnp.concatenate([task_h, vm_h], axis=0)

    src = jnp.concatenate([compatibilities[0], task_dependencies[0]])
    dst = jnp.concatenate([compatibilities[1] + NT, task_dependencies[1]])
    pad_e = EP - E
    src_p = jnp.concatenate([src, jnp.zeros((pad_e,), jnp.int32)])
    dst_p = jnp.concatenate([dst, jnp.full((pad_e,), N, jnp.int32)])
    dst3 = dst_p.reshape(NW, KCH, C)
    zrows = jnp.zeros((ZR, 128), jnp.float32)

    def srcq(q_chunks):
        off = (jnp.arange(q_chunks, dtype=jnp.int32) * N)[:, None]
        return (src_p[None, :] + off).reshape(q_chunks, NW, KCH, C)

    parts1 = _sc_segsum(node_x, srcq(1), dst3, zrows, 1)
    h1q = _conv1(node_x, parts1[:, 0], p['gin_W1a'], r2(p['gin_b1a']),
                 p['gin_W1b'], r2(p['gin_b1b']))
    parts2 = _sc_segsum(h1q.reshape(4 * N, 128), srcq(4), dst3, zrows, 4)
    y = _conv2_proj(h1q, parts2, p['gin_W2a'], r2(p['gin_b2a']),
                    p['gin_W2b'], r2(p['gin_b2b']), p['gin_W3a'])
    parts3 = _sc_segsum(y, srcq(1), dst3, zrows, 1)
    ne, gemb = _conv3_out(y, parts3[:, 0], r2(p['gin_b3a']),
                          p['gin_W3b'], r2(p['gin_b3b']))
    ee = _sc_edge_embed(ne, src_p.reshape(NW, KCH, C), dst3)
    return ne, ee, gemb
